# baseline (speedup 1.0000x reference)
import functools

import jax
import jax.numpy as jnp
from jax import lax
from jax.experimental import pallas as pl
from jax.experimental.pallas import tpu as pltpu
from jax.experimental.pallas import tpu_sc as plsc

N_TAIL = 10000
N_TAR = 10000
D = 256
E = 160000

NC = 2
NS = 16
K = 64
KT = 3 * K
HALF = N_TAR // NC
ACC_ROWS = 5008
EPT = E // NS
T = 54
GSENT = -1
ROWS_PER_TILE = 312


def _proj_body(h_ref, w_ref, b_ref, o_ref):
    x = lax.dot_general(
        h_ref[...], w_ref[...], (((1,), (1,)), ((), ())),
        preferred_element_type=jnp.float32,
        precision=lax.Precision.HIGHEST)
    o_ref[...] = x + b_ref[...]


def _project(h, w, b):
    return pl.pallas_call(
        _proj_body,
        grid=(10,),
        in_specs=[
            pl.BlockSpec((N_TAIL // 10, D), lambda i: (i, 0)),
            pl.BlockSpec((D, D), lambda i: (0, 0)),
            pl.BlockSpec((1, D), lambda i: (0, 0)),
        ],
        out_specs=pl.BlockSpec((N_TAIL // 10, D), lambda i: (i, 0)),
        out_shape=jax.ShapeDtypeStruct((N_TAIL, D), jnp.float32),
    )(h, w, b)


def _sc_body(ea, et, xa, xt, out,
             rsp0, rdp0, rsp1, rdp1,
             fs0, fd0, fs1, fd1, fs2, fd2,
             rows0, rows1, rows2, acc,
             si0, si1, sg0, sg1, sg2, ss):
    c = lax.axis_index("c")
    s = lax.axis_index("s")
    lo = c * HALF

    def zrow(i, _):
        r = i // 16
        j = i % 16
        rows0[r, pl.ds(j * 16, 16)] = jnp.zeros((16,), jnp.float32)
        return 0
    lax.fori_loop(0, 16 * 16, zrow, 0)

    zbase = s * (ACC_ROWS // NS)
    def zacc(i, _):
        pltpu.sync_copy(rows0.at[pl.ds(0, 16)],
                        acc.at[pl.ds(zbase + i * 16, 16)])
        return 0
    lax.fori_loop(0, 19, zacc, 0)
    pltpu.sync_copy(rows0.at[pl.ds(0, 9)],
                    acc.at[pl.ds(zbase + 304, 9)])

    plsc.subcore_barrier()

    elem_base = s * EPT
    F = ((fs0, fd0, rows0, sg0), (fs1, fd1, rows1, sg1),
         (fs2, fd2, rows2, sg2))
    SETS = ((rsp0, rdp0, si0), (rsp1, rdp1, si1))

    def phase(e2d, table):
        def idx_base(t):
            return jnp.minimum(elem_base + t * KT, E - KT)

        def fire_idx(t, rsp, rdp, si):
            base = idx_base(t)
            pltpu.async_copy(e2d.at[pl.ds(base, KT)], rsp, si)
            pltpu.async_copy(e2d.at[pl.ds(E + base, KT)], rdp, si)

        def wait_idx(t, rsp, rdp, si):
            base = idx_base(t)
            pltpu.make_async_copy(e2d.at[pl.ds(base, KT)], rsp, si).wait()
            pltpu.make_async_copy(
                e2d.at[pl.ds(E + base, KT)], rdp, si).wait()

        def remap(t, rsp, rdp, i, fs, fd):
            pos0 = idx_base(t) + i * K
            lo_t = jnp.minimum(elem_base + t * KT, elem_base + EPT)
            hi_t = jnp.minimum(elem_base + (t + 1) * KT, elem_base + EPT)
            for j in range(K // 16):
                slr = pl.ds(i * K + j * 16, 16)
                slf = pl.ds(j * 16, 16)
                pos = pos0 + j * 16 + lax.iota(jnp.int32, 16)
                sv = rsp[slr]
                dv = rdp[slr]
                ok = ((dv >= lo) & (dv < lo + HALF)
                      & (pos >= lo_t) & (pos < hi_t))
                fs[slf] = jnp.where(ok, sv, GSENT)
                fd[slf] = jnp.where(ok, dv - lo, HALF)

        def fire_gather(fs, rows, sg):
            pltpu.async_copy(
                table.at[plsc.Indices(fs, ignored_value=GSENT)], rows, sg)

        def wait_gather(fs, rows, sg):
            pltpu.make_async_copy(
                table.at[plsc.Indices(fs, ignored_value=GSENT)], rows,
                sg).wait()

        def scatter(fd, rows):
            pltpu.async_copy(
                rows, acc.at[plsc.Indices(fd, ignored_value=HALF)], ss,
                add=True).wait()

        fire_idx(0, rsp0, rdp0, si0)
        fire_idx(1, rsp1, rdp1, si1)
        wait_idx(0, rsp0, rdp0, si0)
        for i in range(3):
            fs, fd, rows, sg = F[i]
            remap(0, rsp0, rdp0, i, fs, fd)
            fire_gather(fs, rows, sg)

        def two_triples(w, _):
            for par in (0, 1):
                t = 2 * w + par
                rsp_c, rdp_c, si_c = SETS[par]
                rsp_n, rdp_n, si_n = SETS[1 - par]
                tn = jnp.minimum(t + 1, T - 1)
                tn2 = jnp.minimum(t + 2, T - 1)
                wait_idx(tn, rsp_n, rdp_n, si_n)
                for i in range(3):
                    fs, fd, rows, sg = F[i]
                    wait_gather(fs, rows, sg)
                    scatter(fd, rows)
                    remap(tn, rsp_n, rdp_n, i, fs, fd)
                    fire_gather(fs, rows, sg)
                fire_idx(tn2, rsp_c, rdp_c, si_c)
            return 0

        lax.fori_loop(0, T // 2, two_triples, 0)

        for i in range(3):
            fs, fd, rows, sg = F[i]
            wait_gather(fs, rows, sg)
        wait_idx(T - 1, rsp1, rdp1, si1)

    phase(ea, xa)
    phase(et, xt)

    plsc.subcore_barrier()

    wbase = s * ROWS_PER_TILE
    for t in range(6):
        n = 52
        r0 = wbase + t * n
        pltpu.sync_copy(acc.at[pl.ds(r0, n)], rows0.at[pl.ds(0, n)])
        pltpu.sync_copy(rows0.at[pl.ds(0, n)], out.at[pl.ds(lo + r0, n)])

    @pl.when(s == 0)
    def _():
        r0 = NS * ROWS_PER_TILE
        n = HALF - r0
        pltpu.sync_copy(acc.at[pl.ds(r0, n)], rows0.at[pl.ds(0, n)])
        pltpu.sync_copy(rows0.at[pl.ds(0, n)], out.at[pl.ds(lo + r0, n)])


_sc_agg = functools.partial(
    pl.kernel,
    out_type=jax.ShapeDtypeStruct((N_TAR, D), jnp.float32),
    mesh=plsc.VectorSubcoreMesh(core_axis_name="c", subcore_axis_name="s"),
    scratch_types=[
        pltpu.VMEM((KT,), jnp.int32),
        pltpu.VMEM((KT,), jnp.int32),
        pltpu.VMEM((KT,), jnp.int32),
        pltpu.VMEM((KT,), jnp.int32),
        pltpu.VMEM((K,), jnp.int32),
        pltpu.VMEM((K,), jnp.int32),
        pltpu.VMEM((K,), jnp.int32),
        pltpu.VMEM((K,), jnp.int32),
        pltpu.VMEM((K,), jnp.int32),
        pltpu.VMEM((K,), jnp.int32),
        pltpu.VMEM((K, D), jnp.float32),
        pltpu.VMEM((K, D), jnp.float32),
        pltpu.VMEM((K, D), jnp.float32),
        pltpu.VMEM_SHARED((ACC_ROWS, D), jnp.float32),
        pltpu.SemaphoreType.DMA,
        pltpu.SemaphoreType.DMA,
        pltpu.SemaphoreType.DMA,
        pltpu.SemaphoreType.DMA,
        pltpu.SemaphoreType.DMA,
        pltpu.SemaphoreType.DMA,
    ],
    compiler_params=pltpu.CompilerParams(use_tc_tiling_on_sc=False),
)(_sc_body)


@jax.jit
def kernel(h_author, h_term, h_paper, edge_index_author, edge_index_term,
           W_author, b_author, W_term, b_term):
    xa = _project(h_author, W_author, b_author[None, :])
    xt = _project(h_term, W_term, b_term[None, :])
    return _sc_agg(edge_index_author.reshape(2 * E),
                   edge_index_term.reshape(2 * E), xa, xt)


# speedup vs baseline: 1.0094x; 1.0094x over previous
import functools

import jax
import jax.numpy as jnp
from jax import lax
from jax.experimental import pallas as pl
from jax.experimental.pallas import tpu as pltpu
from jax.experimental.pallas import tpu_sc as plsc

N_TAIL = 10000
N_TAR = 10000
D = 256
E = 160000

NC = 2
NS = 16
K = 64
KT = 3 * K
HALF = N_TAR // NC
ACC_ROWS = 5008
EPT = E // NS
T = 54
GSENT = -1
ROWS_PER_TILE = 312


def _proj_body(h_ref, w_ref, b_ref, o_ref):
    x = lax.dot_general(
        h_ref[...], w_ref[...], (((1,), (1,)), ((), ())),
        preferred_element_type=jnp.float32,
        precision=lax.Precision.HIGHEST)
    o_ref[...] = x + b_ref[...]


def _project(h, w, b):
    return pl.pallas_call(
        _proj_body,
        grid=(10,),
        in_specs=[
            pl.BlockSpec((N_TAIL // 10, D), lambda i: (i, 0)),
            pl.BlockSpec((D, D), lambda i: (0, 0)),
            pl.BlockSpec((1, D), lambda i: (0, 0)),
        ],
        out_specs=pl.BlockSpec((N_TAIL // 10, D), lambda i: (i, 0)),
        out_shape=jax.ShapeDtypeStruct((N_TAIL, D), jnp.float32),
    )(h, w, b)


def _sc_body(ea, et, xa, xt, out,
             rsp0, rdp0, rsp1, rdp1,
             fs0, fs1, fs2,
             fd0, fd1, fd2, fd3, fd4, fd5,
             rows0, rows1, rows2, acc,
             si0, si1, sg0, sg1, sg2, ss0, ss1, ss2):
    c = lax.axis_index("c")
    s = lax.axis_index("s")
    lo = c * HALF

    def zrow(i, _):
        r = i // 16
        j = i % 16
        rows0[r, pl.ds(j * 16, 16)] = jnp.zeros((16,), jnp.float32)
        return 0
    lax.fori_loop(0, 16 * 16, zrow, 0)

    zbase = s * (ACC_ROWS // NS)
    def zacc(i, _):
        pltpu.sync_copy(rows0.at[pl.ds(0, 16)],
                        acc.at[pl.ds(zbase + i * 16, 16)])
        return 0
    lax.fori_loop(0, 19, zacc, 0)
    pltpu.sync_copy(rows0.at[pl.ds(0, 9)],
                    acc.at[pl.ds(zbase + 304, 9)])

    plsc.subcore_barrier()

    elem_base = s * EPT
    F = ((fs0, rows0, sg0, ss0), (fs1, rows1, sg1, ss1),
         (fs2, rows2, sg2, ss2))
    FD = ((fd0, fd1, fd2), (fd3, fd4, fd5))
    SETS = ((rsp0, rdp0, si0), (rsp1, rdp1, si1))

    def phase(e2d, table):
        def idx_base(t):
            return jnp.minimum(elem_base + t * KT, E - KT)

        def fire_idx(t, rsp, rdp, si):
            base = idx_base(t)
            pltpu.async_copy(e2d.at[pl.ds(base, KT)], rsp, si)
            pltpu.async_copy(e2d.at[pl.ds(E + base, KT)], rdp, si)

        def wait_idx(t, rsp, rdp, si):
            base = idx_base(t)
            pltpu.make_async_copy(e2d.at[pl.ds(base, KT)], rsp, si).wait()
            pltpu.make_async_copy(
                e2d.at[pl.ds(E + base, KT)], rdp, si).wait()

        def remap(t, rsp, rdp, i, fs, fd):
            pos0 = idx_base(t) + i * K
            lo_t = jnp.minimum(elem_base + t * KT, elem_base + EPT)
            hi_t = jnp.minimum(elem_base + (t + 1) * KT, elem_base + EPT)
            for j in range(K // 16):
                slr = pl.ds(i * K + j * 16, 16)
                slf = pl.ds(j * 16, 16)
                pos = pos0 + j * 16 + lax.iota(jnp.int32, 16)
                sv = rsp[slr]
                dv = rdp[slr]
                ok = ((dv >= lo) & (dv < lo + HALF)
                      & (pos >= lo_t) & (pos < hi_t))
                fs[slf] = jnp.where(ok, sv, GSENT)
                fd[slf] = jnp.where(ok, dv - lo, HALF)

        def fire_gather(fs, rows, sg):
            pltpu.async_copy(
                table.at[plsc.Indices(fs, ignored_value=GSENT)], rows, sg)

        def wait_gather(fs, rows, sg):
            pltpu.make_async_copy(
                table.at[plsc.Indices(fs, ignored_value=GSENT)], rows,
                sg).wait()

        def fire_scatter(fd, rows, ss):
            pltpu.async_copy(
                rows, acc.at[plsc.Indices(fd, ignored_value=HALF)], ss,
                add=True)

        def wait_scatter(fd, rows, ss):
            pltpu.make_async_copy(
                rows, acc.at[plsc.Indices(fd, ignored_value=HALF)],
                ss).wait()

        fire_idx(0, rsp0, rdp0, si0)
        fire_idx(1, rsp1, rdp1, si1)
        wait_idx(0, rsp0, rdp0, si0)
        for i in range(3):
            fs, rows, sg, ss = F[i]
            remap(0, rsp0, rdp0, i, fs, FD[0][i])
            fire_gather(fs, rows, sg)

        def two_triples(w, _):
            for par in (0, 1):
                t = 2 * w + par
                rsp_c, rdp_c, si_c = SETS[par]
                rsp_n, rdp_n, si_n = SETS[1 - par]
                tn = jnp.minimum(t + 1, T - 1)
                tn2 = jnp.minimum(t + 2, T - 1)
                wait_idx(tn, rsp_n, rdp_n, si_n)
                for i in range(3):
                    fs, rows, sg, ss = F[i]
                    fd = FD[par][i]
                    wait_gather(fs, rows, sg)
                    fire_scatter(fd, rows, ss)
                    remap(tn, rsp_n, rdp_n, i, fs, FD[1 - par][i])
                    wait_scatter(fd, rows, ss)
                    fire_gather(fs, rows, sg)
                fire_idx(tn2, rsp_c, rdp_c, si_c)
            return 0

        lax.fori_loop(0, T // 2, two_triples, 0)

        for i in range(3):
            fs, rows, sg, ss = F[i]
            wait_gather(fs, rows, sg)
        wait_idx(T - 1, rsp1, rdp1, si1)

    phase(ea, xa)
    phase(et, xt)

    plsc.subcore_barrier()

    wbase = s * ROWS_PER_TILE
    for t in range(6):
        n = 52
        r0 = wbase + t * n
        pltpu.sync_copy(acc.at[pl.ds(r0, n)], rows0.at[pl.ds(0, n)])
        pltpu.sync_copy(rows0.at[pl.ds(0, n)], out.at[pl.ds(lo + r0, n)])

    @pl.when(s == 0)
    def _():
        r0 = NS * ROWS_PER_TILE
        n = HALF - r0
        pltpu.sync_copy(acc.at[pl.ds(r0, n)], rows0.at[pl.ds(0, n)])
        pltpu.sync_copy(rows0.at[pl.ds(0, n)], out.at[pl.ds(lo + r0, n)])


_sc_agg = functools.partial(
    pl.kernel,
    out_type=jax.ShapeDtypeStruct((N_TAR, D), jnp.float32),
    mesh=plsc.VectorSubcoreMesh(core_axis_name="c", subcore_axis_name="s"),
    scratch_types=[
        pltpu.VMEM((KT,), jnp.int32),
        pltpu.VMEM((KT,), jnp.int32),
        pltpu.VMEM((KT,), jnp.int32),
        pltpu.VMEM((KT,), jnp.int32),
        pltpu.VMEM((K,), jnp.int32),
        pltpu.VMEM((K,), jnp.int32),
        pltpu.VMEM((K,), jnp.int32),
        pltpu.VMEM((K,), jnp.int32),
        pltpu.VMEM((K,), jnp.int32),
        pltpu.VMEM((K,), jnp.int32),
        pltpu.VMEM((K,), jnp.int32),
        pltpu.VMEM((K,), jnp.int32),
        pltpu.VMEM((K,), jnp.int32),
        pltpu.VMEM((K, D), jnp.float32),
        pltpu.VMEM((K, D), jnp.float32),
        pltpu.VMEM((K, D), jnp.float32),
        pltpu.VMEM_SHARED((ACC_ROWS, D), jnp.float32),
        pltpu.SemaphoreType.DMA,
        pltpu.SemaphoreType.DMA,
        pltpu.SemaphoreType.DMA,
        pltpu.SemaphoreType.DMA,
        pltpu.SemaphoreType.DMA,
        pltpu.SemaphoreType.DMA,
        pltpu.SemaphoreType.DMA,
        pltpu.SemaphoreType.DMA,
    ],
    compiler_params=pltpu.CompilerParams(use_tc_tiling_on_sc=False),
)(_sc_body)


@jax.jit
def kernel(h_author, h_term, h_paper, edge_index_author, edge_index_term,
           W_author, b_author, W_term, b_term):
    xa = _project(h_author, W_author, b_author[None, :])
    xt = _project(h_term, W_term, b_term[None, :])
    return _sc_agg(edge_index_author.reshape(2 * E),
                   edge_index_term.reshape(2 * E), xa, xt)
